# baseline (speedup 1.0000x reference)
import numpy as np
import jax
import jax.numpy as jnp
from jax import lax
from jax.experimental import pallas as pl

_C = 256
_T = 4000
_B, _H, _W = 16, 64, 64
_P = _H * _W
_LOG_EPS = float(np.log(1e-30))
_LOG_C = float(np.log(_C))
_PBLK = 2048


def _schedules():
    s = 0.008
    steps = _T + 1
    x = np.linspace(0, steps, steps)
    ac = np.cos(((x / steps) + s) / (1 + s) * np.pi * 0.5) ** 2
    ac = ac / ac[0]
    al = np.clip(ac[1:] / ac[:-1], a_min=0.001, a_max=1.0).astype(np.float64)
    la = np.log(al)
    lca = np.cumsum(la)

    def l1m(a):
        return np.log(1.0 - np.exp(a) + 1e-40)

    return (la.astype(np.float32), l1m(la).astype(np.float32),
            lca.astype(np.float32), l1m(lca).astype(np.float32))


_LA_NP, _L1A_NP, _LCA_NP, _L1CA_NP = _schedules()


def _log_add_exp(a, b):
    m = jnp.maximum(a, b)
    return m + jnp.log(jnp.exp(a - m) + jnp.exp(b - m))


def _argmax_body(x_ref, o_ref):
    v = x_ref[0]
    cio = lax.broadcasted_iota(jnp.int32, v.shape, 0)
    o_ref[0] = jnp.sum(jnp.where(v > -1.0, cio, 0), axis=0, keepdims=True)


def _threefry_bits(cnt_u32):
    ks0 = jnp.uint32(0)
    ks1 = jnp.uint32(42)
    ks2 = jnp.uint32(0x1BD11BF0)
    ks = (ks0, ks1, ks2)
    rot = ((13, 15, 26, 6), (17, 29, 16, 24))
    x0 = jnp.zeros_like(cnt_u32) + ks0
    x1 = cnt_u32 + ks1
    for g in range(5):
        for r in rot[g % 2]:
            x0 = x0 + x1
            x1 = ((x1 << r) | (x1 >> (32 - r))) ^ x0
        x0 = x0 + ks[(g + 1) % 3]
        x1 = x1 + ks[(g + 2) % 3] + jnp.uint32(g + 1)
    return x0 ^ x1


def _sample_body(taps_ref, x0_ref, bias_ref, sc_ref, w_ref, o_ref):
    i = pl.program_id(0)
    j = pl.program_id(1)
    w = w_ref[...]
    for hb in range(2):
        taps = taps_ref[hb, 0]
        conv = jnp.dot(w, taps, preferred_element_type=jnp.float32)
        x0r = x0_ref[hb, 0]
        ci = lax.broadcasted_iota(jnp.int32, conv.shape, 0)
        oh = ci == x0r
        logits = conv + bias_ref[hb, 0] + jnp.where(oh, 1.0, 0.0)
        m = jnp.max(logits, axis=0, keepdims=True)
        e = jnp.exp(logits - m)
        s = jnp.sum(e, axis=0, keepdims=True)
        ls = (logits - m) - jnp.log(s)
        scrow = sc_ref[hb, 0]
        lc = scrow[:, 0:1]
        l1k = scrow[:, 1:2]
        t0 = scrow[:, 2:3]
        vm = scrow[:, 3:4]
        vo = scrow[:, 4:5]
        a = ls + lc
        m2 = jnp.maximum(a, l1k)
        le = m2 + jnp.log(jnp.exp(a - m2) + jnp.exp(l1k - m2))
        le = jnp.where(t0 > 0.5, ls, le)
        unn = le + jnp.where(oh, vm, vo)
        base = hb * 8388608 + i * 1048576 + j * _PBLK
        cnt = ci * _P + lax.broadcasted_iota(jnp.int32, conv.shape, 1) + base
        bits = _threefry_bits(cnt.astype(jnp.uint32))
        fb = lax.bitcast_convert_type((bits >> 9) | jnp.uint32(0x3F800000),
                                      jnp.float32)
        u = fb - 1.0
        g = -jnp.log(-jnp.log(u + 1e-30) + 1e-30)
        tot = unn + g
        mt = jnp.max(tot, axis=0, keepdims=True)
        cand = jnp.where(tot == mt, ci, _C)
        samp = jnp.min(cand, axis=0, keepdims=True)
        o_ref[hb, 0] = jnp.where(ci == samp, 0.0, _LOG_EPS)


def kernel(log_x, t, W_dn, b_dn):
    lx = log_x.reshape(_B, _C, _P)
    x0i = pl.pallas_call(
        _argmax_body,
        grid=(_B,),
        in_specs=[pl.BlockSpec((1, _C, _P), lambda b: (b, 0, 0))],
        out_specs=pl.BlockSpec((1, 1, _P), lambda b: (b, 0, 0)),
        out_shape=jax.ShapeDtypeStruct((_B, 1, _P), jnp.int32),
    )(lx)

    xf = x0i.reshape(_B, _H, _W).astype(jnp.float32) / (_C - 1)
    xp = jnp.pad(xf, ((0, 0), (1, 1), (1, 1)))
    taps = jnp.stack([xp[:, dy:dy + _H, dx:dx + _W]
                      for dy in range(3) for dx in range(3)], axis=1)
    taps = taps.reshape(_B, 9, _P)
    taps = jnp.pad(taps, ((0, 0), (0, 7), (0, 0))).reshape(2, 8, 16, _P)

    w9 = jnp.pad(W_dn.reshape(_C, 9), ((0, 0), (0, 7)))

    tf = t.astype(jnp.float32)
    bias = (b_dn[None, :] + (tf / _T)[:, None]).reshape(2, 8, _C, 1)

    la = jnp.asarray(_LA_NP)
    l1a = jnp.asarray(_L1A_NP)
    lca = jnp.asarray(_LCA_NP)
    l1ca = jnp.asarray(_L1CA_NP)
    tm1 = jnp.maximum(t - 1, 0)
    lc = jnp.take(lca, tm1)
    l1k = jnp.take(l1ca, tm1) - _LOG_C
    lat = jnp.take(la, t)
    l1at = jnp.take(l1a, t) - _LOG_C
    vm = _log_add_exp(lat, l1at)
    vo = _log_add_exp(_LOG_EPS + lat, l1at)
    t0 = (t == 0).astype(jnp.float32)
    sc = jnp.stack([lc, l1k, t0, vm, vo], axis=1)
    sc = jnp.pad(sc, ((0, 0), (0, 123))).reshape(2, 8, 1, 128)

    x0r = x0i.reshape(2, 8, 1, _P)

    out = pl.pallas_call(
        _sample_body,
        grid=(8, _P // _PBLK),
        in_specs=[
            pl.BlockSpec((2, 1, 16, _PBLK), lambda i, j: (0, i, 0, j)),
            pl.BlockSpec((2, 1, 1, _PBLK), lambda i, j: (0, i, 0, j)),
            pl.BlockSpec((2, 1, _C, 1), lambda i, j: (0, i, 0, 0)),
            pl.BlockSpec((2, 1, 1, 128), lambda i, j: (0, i, 0, 0)),
            pl.BlockSpec((_C, 16), lambda i, j: (0, 0)),
        ],
        out_specs=pl.BlockSpec((2, 1, _C, _PBLK), lambda i, j: (0, i, 0, j)),
        out_shape=jax.ShapeDtypeStruct((2, 8, _C, _P), jnp.float32),
    )(taps, x0r, bias, sc, w9)
    return out.reshape(_B, _C, 1, _H, _W)
